# baseline (speedup 1.0000x reference)
import jax
import jax.numpy as jnp
from jax.experimental import pallas as pl
from jax.experimental.pallas import tpu as pltpu

_VMEM = 64 * 1024 * 1024


def _mm_body(a_ref, w_ref, b_ref, o_ref):
    y = jnp.dot(a_ref[...], w_ref[...], preferred_element_type=jnp.float32)
    o_ref[...] = jnp.maximum(y + b_ref[...], 0.0).astype(o_ref.dtype)


def _mm_body_norelu(a_ref, w_ref, b_ref, o_ref):
    y = jnp.dot(a_ref[...], w_ref[...], preferred_element_type=jnp.float32)
    o_ref[...] = (y + b_ref[...]).astype(o_ref.dtype)


def _mm_body_res(a_ref, w_ref, b_ref, r_ref, o_ref):
    y = jnp.dot(a_ref[...], w_ref[...], preferred_element_type=jnp.float32)
    y = y + b_ref[...] + r_ref[...].astype(jnp.float32)
    o_ref[...] = jnp.maximum(y, 0.0).astype(o_ref.dtype)


def _pick_tm(m):
    for c in (512, 448, 256, 224, 128, 112, 64, 56, 32, 16, 8):
        if m % c == 0 and m // c >= 8:
            return c
    for c in (512, 448, 256, 224, 128, 112, 64, 56, 32, 16, 8):
        if m % c == 0:
            return c
    return m


def _matmul(a, w, b, residual=None, relu=True):
    m, k = a.shape
    kp, n = w.shape
    if kp != k:
        a = jnp.pad(a, ((0, 0), (0, kp - k)))
    tm = _pick_tm(m)
    grid = (m // tm,)
    in_specs = [
        pl.BlockSpec((tm, kp), lambda i: (i, 0)),
        pl.BlockSpec((kp, n), lambda i: (0, 0)),
        pl.BlockSpec((1, n), lambda i: (0, 0)),
    ]
    inputs = [a, w, b]
    if residual is not None:
        body = _mm_body_res
        in_specs.append(pl.BlockSpec((tm, n), lambda i: (i, 0)))
        inputs.append(residual)
    else:
        body = _mm_body if relu else _mm_body_norelu
    return pl.pallas_call(
        body,
        out_shape=jax.ShapeDtypeStruct((m, n), jnp.bfloat16),
        grid=grid,
        in_specs=in_specs,
        out_specs=pl.BlockSpec((tm, n), lambda i: (i, 0)),
        compiler_params=pltpu.CompilerParams(
            dimension_semantics=("parallel",), vmem_limit_bytes=_VMEM),
    )(*inputs)


def _c3s1_body(x0_ref, x1_ref, x2_ref, w_ref, b_ref, o_ref):
    n, wo, co = o_ref.shape
    c = x0_ref.shape[-1]
    acc = jnp.zeros((n * wo, co), jnp.float32)
    for di, xr in enumerate((x0_ref, x1_ref, x2_ref)):
        xv = xr[...]
        for dj in range(3):
            a = xv[:, dj:dj + wo, :].reshape(n * wo, c)
            acc = acc + jnp.dot(a, w_ref[di * 3 + dj],
                                preferred_element_type=jnp.float32)
    y = jnp.maximum(acc + b_ref[...], 0.0)
    o_ref[...] = y.reshape(n, wo, co).astype(o_ref.dtype)


def _conv3x3_s1(x, w9, b):
    n, h, w, c = x.shape
    co = w9.shape[2]
    xp = jnp.pad(x, ((0, 0), (1, 1), (1, 1), (0, 0)))
    wp = w + 2

    def xs(di):
        return pl.BlockSpec((n, None, wp, c), lambda r, di=di: (0, r + di, 0, 0))

    return pl.pallas_call(
        _c3s1_body,
        out_shape=jax.ShapeDtypeStruct((n, h, w, co), x.dtype),
        grid=(h,),
        in_specs=[xs(0), xs(1), xs(2),
                  pl.BlockSpec((9, c, co), lambda r: (0, 0, 0)),
                  pl.BlockSpec((1, co), lambda r: (0, 0))],
        out_specs=pl.BlockSpec((n, None, w, co), lambda r: (0, r, 0, 0)),
        compiler_params=pltpu.CompilerParams(
            dimension_semantics=("parallel",), vmem_limit_bytes=_VMEM),
    )(xp, xp, xp, w9, b)


def _c3s2_body(e0_ref, e1_ref, e2_ref, q0_ref, q1_ref, q2_ref,
               w_ref, b_ref, o_ref):
    n, wo, co = o_ref.shape
    c = e0_ref.shape[-1]
    acc = jnp.zeros((n * wo, co), jnp.float32)
    for di, (er, qr) in enumerate(((e0_ref, q0_ref), (e1_ref, q1_ref),
                                   (e2_ref, q2_ref))):
        ev = er[...]
        qv = qr[...]
        taps = (ev[:, 0:wo, :], qv[:, 0:wo, :], ev[:, 1:1 + wo, :])
        for dj in range(3):
            acc = acc + jnp.dot(taps[dj].reshape(n * wo, c),
                                w_ref[di * 3 + dj],
                                preferred_element_type=jnp.float32)
    y = jnp.maximum(acc + b_ref[...], 0.0)
    o_ref[...] = y.reshape(n, wo, co).astype(o_ref.dtype)


def _conv3x3_s2(x, wflat, b):
    n, h, w, c = x.shape
    co = wflat.shape[1]
    w9 = wflat[:9 * c].reshape(9, c, co)
    ho, wo = h // 2, w // 2
    xp = jnp.pad(x, ((0, 0), (1, 1), (1, 1), (0, 0)))
    xe = xp[:, :, 0::2, :]
    xq = xp[:, :, 1::2, :]
    we_ = xe.shape[2]
    wq_ = xq.shape[2]

    def es(di):
        return pl.BlockSpec((n, None, we_, c),
                            lambda r, di=di: (0, 2 * r + di, 0, 0))

    def qs(di):
        return pl.BlockSpec((n, None, wq_, c),
                            lambda r, di=di: (0, 2 * r + di, 0, 0))

    return pl.pallas_call(
        _c3s2_body,
        out_shape=jax.ShapeDtypeStruct((n, ho, wo, co), x.dtype),
        grid=(ho,),
        in_specs=[es(0), es(1), es(2), qs(0), qs(1), qs(2),
                  pl.BlockSpec((9, c, co), lambda r: (0, 0, 0)),
                  pl.BlockSpec((1, co), lambda r: (0, 0))],
        out_specs=pl.BlockSpec((n, None, wo, co), lambda r: (0, r, 0, 0)),
        compiler_params=pltpu.CompilerParams(
            dimension_semantics=("parallel",), vmem_limit_bytes=_VMEM),
    )(xe, xe, xe, xq, xq, xq, w9, b)


def _stem_body(x0_ref, x1_ref, x2_ref, x3_ref, x4_ref, x5_ref, x6_ref,
               w_ref, b_ref, o_ref):
    n, wo, co = o_ref.shape
    kc = x0_ref.shape[-1]
    acc = jnp.zeros((n * wo, co), jnp.float32)
    for di, xr in enumerate((x0_ref, x1_ref, x2_ref, x3_ref, x4_ref,
                             x5_ref, x6_ref)):
        acc = acc + jnp.dot(xr[...].reshape(n * wo, kc), w_ref[di],
                            preferred_element_type=jnp.float32)
    y = jnp.maximum(acc + b_ref[...], 0.0)
    o_ref[...] = y.reshape(n, wo, co).astype(o_ref.dtype)


def _stem_conv(x_nhwc, stem_w, stem_b):
    n, h, w, _ = x_nhwc.shape
    ho = wo = h // 2
    co = stem_w.shape[1]
    xp = jnp.pad(x_nhwc, ((0, 0), (3, 3), (3, 3), (0, 0)))
    hp = h + 6
    cols = [xp[:, :, j:j + 2 * wo - 1:2, :] for j in range(7)]
    cols.append(jnp.zeros((n, hp, wo, 3), x_nhwc.dtype))
    cp = jnp.concatenate(cols, axis=-1)
    wt = jnp.pad(stem_w[:147].reshape(7, 21, co), ((0, 0), (0, 3), (0, 0)))

    def xs(di):
        return pl.BlockSpec((n, None, wo, 24),
                            lambda r, di=di: (0, 2 * r + di, 0, 0))

    return pl.pallas_call(
        _stem_body,
        out_shape=jax.ShapeDtypeStruct((n, ho, wo, co), x_nhwc.dtype),
        grid=(ho,),
        in_specs=[xs(0), xs(1), xs(2), xs(3), xs(4), xs(5), xs(6),
                  pl.BlockSpec((7, 24, co), lambda r: (0, 0, 0)),
                  pl.BlockSpec((1, co), lambda r: (0, 0))],
        out_specs=pl.BlockSpec((n, None, wo, co), lambda r: (0, r, 0, 0)),
        compiler_params=pltpu.CompilerParams(
            dimension_semantics=("parallel",), vmem_limit_bytes=_VMEM),
    )(cp, cp, cp, cp, cp, cp, cp, wt, stem_b)


def _maxpool_body(e0_ref, e1_ref, e2_ref, q0_ref, q1_ref, q2_ref, o_ref):
    n, wo, c = o_ref.shape
    res = None
    for er, qr in ((e0_ref, q0_ref), (e1_ref, q1_ref), (e2_ref, q2_ref)):
        ev = er[...]
        qv = qr[...]
        rm = jnp.maximum(jnp.maximum(ev[:, 0:wo, :], qv[:, 0:wo, :]),
                         ev[:, 1:1 + wo, :])
        res = rm if res is None else jnp.maximum(res, rm)
    o_ref[...] = res


def _maxpool_3x3_s2(x):
    n, h, w, c = x.shape
    ho = (h - 1) // 2 + 1
    wo = (w - 1) // 2 + 1
    xp = jnp.pad(x, ((0, 0), (1, 1), (1, 1), (0, 0)),
                 constant_values=-jnp.inf)
    xe = xp[:, :, 0::2, :]
    xq = xp[:, :, 1::2, :]
    we_ = xe.shape[2]
    wq_ = xq.shape[2]

    def es(di):
        return pl.BlockSpec((n, None, we_, c),
                            lambda r, di=di: (0, 2 * r + di, 0, 0))

    def qs(di):
        return pl.BlockSpec((n, None, wq_, c),
                            lambda r, di=di: (0, 2 * r + di, 0, 0))

    return pl.pallas_call(
        _maxpool_body,
        out_shape=jax.ShapeDtypeStruct((n, ho, wo, c), x.dtype),
        grid=(ho,),
        in_specs=[es(0), es(1), es(2), qs(0), qs(1), qs(2)],
        out_specs=pl.BlockSpec((n, None, wo, c), lambda r: (0, r, 0, 0)),
        compiler_params=pltpu.CompilerParams(
            dimension_semantics=("parallel",), vmem_limit_bytes=_VMEM),
    )(xe, xe, xe, xq, xq, xq)


def _avgpool_body(x_ref, o_ref):
    x = x_ref[...].astype(jnp.float32)
    o_ref[...] = jnp.sum(x, axis=1) * (1.0 / x_ref.shape[1])


def _global_avgpool(x):
    n, h, w, c = x.shape
    hw = h * w
    xr = x.reshape(n, hw, c)
    tc = 512 if c % 512 == 0 else c
    return pl.pallas_call(
        _avgpool_body,
        out_shape=jax.ShapeDtypeStruct((n, c), jnp.float32),
        grid=(c // tc,),
        in_specs=[pl.BlockSpec((n, hw, tc), lambda j: (0, 0, j))],
        out_specs=pl.BlockSpec((n, tc), lambda j: (0, j)),
        compiler_params=pltpu.CompilerParams(
            dimension_semantics=("parallel",), vmem_limit_bytes=_VMEM),
    )(xr)


_CFG = ((3, 1), (4, 2), (6, 2), (3, 2))


def _bottleneck(x, c1w, c1b, c2w, c2b, c3w, c3b, downw, downb, stride):
    n, h, w, ci = x.shape
    if downw is not None:
        xs = x[:, ::stride, ::stride, :] if stride > 1 else x
        hs, ws = xs.shape[1], xs.shape[2]
        idn = _matmul(xs.reshape(n * hs * ws, ci), downw, downb, relu=False)
    else:
        idn = None
    y = _matmul(x.reshape(n * h * w, ci), c1w, c1b).reshape(n, h, w, -1)
    if stride == 1:
        y = _conv3x3_s1(y, c2w, c2b)
    else:
        y = _conv3x3_s2(y, c2w, c2b)
    n2, h2, w2, cm = y.shape
    co = c3w.shape[1]
    if idn is None:
        idn = x.reshape(n2 * h2 * w2, co)
    y = _matmul(y.reshape(n2 * h2 * w2, cm), c3w, c3b, residual=idn)
    return y.reshape(n2, h2, w2, co)


def kernel(x, stem_w, stem_b, l0b0_c1_w, l0b0_c1_b, l0b0_c2_w, l0b0_c2_b, l0b0_c3_w, l0b0_c3_b, l0b0_down_w, l0b0_down_b, l0b1_c1_w, l0b1_c1_b, l0b1_c2_w, l0b1_c2_b, l0b1_c3_w, l0b1_c3_b, l0b2_c1_w, l0b2_c1_b, l0b2_c2_w, l0b2_c2_b, l0b2_c3_w, l0b2_c3_b, l1b0_c1_w, l1b0_c1_b, l1b0_c2_w, l1b0_c2_b, l1b0_c3_w, l1b0_c3_b, l1b0_down_w, l1b0_down_b, l1b1_c1_w, l1b1_c1_b, l1b1_c2_w, l1b1_c2_b, l1b1_c3_w, l1b1_c3_b, l1b2_c1_w, l1b2_c1_b, l1b2_c2_w, l1b2_c2_b, l1b2_c3_w, l1b2_c3_b, l1b3_c1_w, l1b3_c1_b, l1b3_c2_w, l1b3_c2_b, l1b3_c3_w, l1b3_c3_b, l2b0_c1_w, l2b0_c1_b, l2b0_c2_w, l2b0_c2_b, l2b0_c3_w, l2b0_c3_b, l2b0_down_w, l2b0_down_b, l2b1_c1_w, l2b1_c1_b, l2b1_c2_w, l2b1_c2_b, l2b1_c3_w, l2b1_c3_b, l2b2_c1_w, l2b2_c1_b, l2b2_c2_w, l2b2_c2_b, l2b2_c3_w, l2b2_c3_b, l2b3_c1_w, l2b3_c1_b, l2b3_c2_w, l2b3_c2_b, l2b3_c3_w, l2b3_c3_b, l2b4_c1_w, l2b4_c1_b, l2b4_c2_w, l2b4_c2_b, l2b4_c3_w, l2b4_c3_b, l2b5_c1_w, l2b5_c1_b, l2b5_c2_w, l2b5_c2_b, l2b5_c3_w, l2b5_c3_b, l3b0_c1_w, l3b0_c1_b, l3b0_c2_w, l3b0_c2_b, l3b0_c3_w, l3b0_c3_b, l3b0_down_w, l3b0_down_b, l3b1_c1_w, l3b1_c1_b, l3b1_c2_w, l3b1_c2_b, l3b1_c3_w, l3b1_c3_b, l3b2_c1_w, l3b2_c1_b, l3b2_c2_w, l3b2_c2_b, l3b2_c3_w, l3b2_c3_b):
    prm = dict(locals())
    xh = jnp.transpose(x, (0, 2, 3, 1)).astype(jnp.bfloat16)
    xh = _stem_conv(xh, stem_w, stem_b)
    xh = _maxpool_3x3_s2(xh)
    for li, (blocks, stride) in enumerate(_CFG):
        for bi in range(blocks):
            s = stride if bi == 0 else 1
            pfx = f"l{li}b{bi}_"
            downw = prm.get(pfx + "down_w")
            downb = prm.get(pfx + "down_b")
            xh = _bottleneck(xh,
                             prm[pfx + "c1_w"], prm[pfx + "c1_b"],
                             prm[pfx + "c2_w"], prm[pfx + "c2_b"],
                             prm[pfx + "c3_w"], prm[pfx + "c3_b"],
                             downw, downb, s)
    return _global_avgpool(xh)


# speedup vs baseline: 4.4051x; 4.4051x over previous
import jax
import jax.numpy as jnp
from jax.experimental import pallas as pl
from jax.experimental.pallas import tpu as pltpu

_VMEM = 64 * 1024 * 1024


def _mm_body(a_ref, w_ref, b_ref, o_ref):
    y = jnp.dot(a_ref[...], w_ref[...], preferred_element_type=jnp.float32)
    o_ref[...] = jnp.maximum(y + b_ref[...], 0.0).astype(o_ref.dtype)


def _mm_body_norelu(a_ref, w_ref, b_ref, o_ref):
    y = jnp.dot(a_ref[...], w_ref[...], preferred_element_type=jnp.float32)
    o_ref[...] = (y + b_ref[...]).astype(o_ref.dtype)


def _mm_body_res(a_ref, w_ref, b_ref, r_ref, o_ref):
    y = jnp.dot(a_ref[...], w_ref[...], preferred_element_type=jnp.float32)
    y = y + b_ref[...] + r_ref[...].astype(jnp.float32)
    o_ref[...] = jnp.maximum(y, 0.0).astype(o_ref.dtype)


def _pick_tm(m):
    for c in (512, 448, 256, 224, 128, 112, 64, 56, 32, 16, 8):
        if m % c == 0 and m // c >= 8:
            return c
    for c in (512, 448, 256, 224, 128, 112, 64, 56, 32, 16, 8):
        if m % c == 0:
            return c
    return m


def _matmul(a, w, b, residual=None, relu=True):
    m, k = a.shape
    kp, n = w.shape
    if kp != k:
        a = jnp.pad(a, ((0, 0), (0, kp - k)))
    tm = _pick_tm(m)
    grid = (m // tm,)
    in_specs = [
        pl.BlockSpec((tm, kp), lambda i: (i, 0)),
        pl.BlockSpec((kp, n), lambda i: (0, 0)),
        pl.BlockSpec((1, n), lambda i: (0, 0)),
    ]
    inputs = [a, w, b]
    if residual is not None:
        body = _mm_body_res
        in_specs.append(pl.BlockSpec((tm, n), lambda i: (i, 0)))
        inputs.append(residual)
    else:
        body = _mm_body if relu else _mm_body_norelu
    return pl.pallas_call(
        body,
        out_shape=jax.ShapeDtypeStruct((m, n), jnp.bfloat16),
        grid=grid,
        in_specs=in_specs,
        out_specs=pl.BlockSpec((tm, n), lambda i: (i, 0)),
        compiler_params=pltpu.CompilerParams(
            dimension_semantics=("parallel",), vmem_limit_bytes=_VMEM),
    )(*inputs)


def _c3s1_body(x0_ref, x1_ref, x2_ref, w_ref, b_ref, o_ref):
    n, wo, co = o_ref.shape
    c = x0_ref.shape[-1]
    acc = jnp.zeros((n * wo, co), jnp.float32)
    for di, xr in enumerate((x0_ref, x1_ref, x2_ref)):
        xv = xr[...]
        for dj in range(3):
            a = xv[:, dj:dj + wo, :].reshape(n * wo, c)
            acc = acc + jnp.dot(a, w_ref[di * 3 + dj],
                                preferred_element_type=jnp.float32)
    y = jnp.maximum(acc + b_ref[...], 0.0)
    o_ref[...] = y.reshape(n, wo, co).astype(o_ref.dtype)


def _conv3x3_s1(x, w9, b):
    n, h, w, c = x.shape
    co = w9.shape[2]
    xp = jnp.pad(x, ((0, 0), (1, 1), (1, 1), (0, 0)))
    wp = w + 2

    def xs(di):
        return pl.BlockSpec((n, None, wp, c), lambda r, di=di: (0, r + di, 0, 0))

    return pl.pallas_call(
        _c3s1_body,
        out_shape=jax.ShapeDtypeStruct((n, h, w, co), x.dtype),
        grid=(h,),
        in_specs=[xs(0), xs(1), xs(2),
                  pl.BlockSpec((9, c, co), lambda r: (0, 0, 0)),
                  pl.BlockSpec((1, co), lambda r: (0, 0))],
        out_specs=pl.BlockSpec((n, None, w, co), lambda r: (0, r, 0, 0)),
        compiler_params=pltpu.CompilerParams(
            dimension_semantics=("parallel",), vmem_limit_bytes=_VMEM),
    )(xp, xp, xp, w9, b)


def _c3s2_body(x0_ref, x1_ref, x2_ref, wa_ref, wb_ref, b_ref, o_ref):
    n, wo, co = o_ref.shape
    c2 = x0_ref.shape[-1]
    c = c2 // 2
    acc = jnp.zeros((n * wo, co), jnp.float32)
    for di, xr in enumerate((x0_ref, x1_ref, x2_ref)):
        xv = xr[...]
        acc = acc + jnp.dot(xv[:, 0:wo, :].reshape(n * wo, c2), wa_ref[di],
                            preferred_element_type=jnp.float32)
        acc = acc + jnp.dot(xv[:, 1:1 + wo, 0:c].reshape(n * wo, c),
                            wb_ref[di], preferred_element_type=jnp.float32)
    y = jnp.maximum(acc + b_ref[...], 0.0)
    o_ref[...] = y.reshape(n, wo, co).astype(o_ref.dtype)


def _conv3x3_s2(x, wflat, b):
    n, h, w, c = x.shape
    co = wflat.shape[1]
    w9 = wflat[:9 * c].reshape(3, 3, c, co)
    wa = w9[:, 0:2].reshape(3, 2 * c, co)
    wb = w9[:, 2]
    ho, wo = h // 2, w // 2
    xp = jnp.pad(x, ((0, 0), (1, 1), (1, 1), (0, 0)))
    wpair = (w + 2) // 2
    xr = xp.reshape(n, h + 2, wpair, 2 * c)

    def xs(di):
        return pl.BlockSpec((n, None, wpair, 2 * c),
                            lambda r, di=di: (0, 2 * r + di, 0, 0))

    return pl.pallas_call(
        _c3s2_body,
        out_shape=jax.ShapeDtypeStruct((n, ho, wo, co), x.dtype),
        grid=(ho,),
        in_specs=[xs(0), xs(1), xs(2),
                  pl.BlockSpec((3, 2 * c, co), lambda r: (0, 0, 0)),
                  pl.BlockSpec((3, c, co), lambda r: (0, 0, 0)),
                  pl.BlockSpec((1, co), lambda r: (0, 0))],
        out_specs=pl.BlockSpec((n, None, wo, co), lambda r: (0, r, 0, 0)),
        compiler_params=pltpu.CompilerParams(
            dimension_semantics=("parallel",), vmem_limit_bytes=_VMEM),
    )(xr, xr, xr, wa, wb, b)


def _stem_body(x0_ref, x1_ref, x2_ref, x3_ref, x4_ref, x5_ref, x6_ref,
               w_ref, b_ref, o_ref):
    n, wo, co = o_ref.shape
    kc = x0_ref.shape[-1]
    acc = jnp.zeros((n * wo, co), jnp.float32)
    for di, xr in enumerate((x0_ref, x1_ref, x2_ref, x3_ref, x4_ref,
                             x5_ref, x6_ref)):
        acc = acc + jnp.dot(xr[...].reshape(n * wo, kc), w_ref[di],
                            preferred_element_type=jnp.float32)
    y = jnp.maximum(acc + b_ref[...], 0.0)
    o_ref[...] = y.reshape(n, wo, co).astype(o_ref.dtype)


def _stem_conv(x_nhwc, stem_w, stem_b):
    n, h, w, _ = x_nhwc.shape
    ho = wo = h // 2
    co = stem_w.shape[1]
    xp = jnp.pad(x_nhwc, ((0, 0), (3, 3), (3, 3), (0, 0)))
    hp = h + 6
    xr = xp.reshape(n, hp, (h + 6) // 2, 6)
    cp = jnp.concatenate([xr[:, :, a:a + wo, :] for a in range(4)], axis=-1)
    wt = jnp.pad(stem_w[:147].reshape(7, 21, co), ((0, 0), (0, 3), (0, 0)))

    def xs(di):
        return pl.BlockSpec((n, None, wo, 24),
                            lambda r, di=di: (0, 2 * r + di, 0, 0))

    return pl.pallas_call(
        _stem_body,
        out_shape=jax.ShapeDtypeStruct((n, ho, wo, co), x_nhwc.dtype),
        grid=(ho,),
        in_specs=[xs(0), xs(1), xs(2), xs(3), xs(4), xs(5), xs(6),
                  pl.BlockSpec((7, 24, co), lambda r: (0, 0, 0)),
                  pl.BlockSpec((1, co), lambda r: (0, 0))],
        out_specs=pl.BlockSpec((n, None, wo, co), lambda r: (0, r, 0, 0)),
        compiler_params=pltpu.CompilerParams(
            dimension_semantics=("parallel",), vmem_limit_bytes=_VMEM),
    )(cp, cp, cp, cp, cp, cp, cp, wt, stem_b)


def _maxpool_body(x0_ref, x1_ref, x2_ref, o_ref):
    n, wo, c = o_ref.shape
    res = None
    for xr in (x0_ref, x1_ref, x2_ref):
        xv = xr[...]
        rm = jnp.maximum(jnp.maximum(xv[:, 0:wo, 0:c], xv[:, 0:wo, c:2 * c]),
                         xv[:, 1:1 + wo, 0:c])
        res = rm if res is None else jnp.maximum(res, rm)
    o_ref[...] = res


def _maxpool_3x3_s2(x):
    n, h, w, c = x.shape
    ho = (h - 1) // 2 + 1
    wo = (w - 1) // 2 + 1
    xp = jnp.pad(x, ((0, 0), (1, 1), (1, 1), (0, 0)),
                 constant_values=-jnp.inf)
    wpair = (w + 2) // 2
    xr = xp.reshape(n, h + 2, wpair, 2 * c)

    def xs(di):
        return pl.BlockSpec((n, None, wpair, 2 * c),
                            lambda r, di=di: (0, 2 * r + di, 0, 0))

    return pl.pallas_call(
        _maxpool_body,
        out_shape=jax.ShapeDtypeStruct((n, ho, wo, c), x.dtype),
        grid=(ho,),
        in_specs=[xs(0), xs(1), xs(2)],
        out_specs=pl.BlockSpec((n, None, wo, c), lambda r: (0, r, 0, 0)),
        compiler_params=pltpu.CompilerParams(
            dimension_semantics=("parallel",), vmem_limit_bytes=_VMEM),
    )(xr, xr, xr)


def _avgpool_body(x_ref, o_ref):
    x = x_ref[...].astype(jnp.float32)
    o_ref[...] = jnp.sum(x, axis=1) * (1.0 / x_ref.shape[1])


def _global_avgpool(x):
    n, h, w, c = x.shape
    hw = h * w
    xr = x.reshape(n, hw, c)
    tc = 512 if c % 512 == 0 else c
    return pl.pallas_call(
        _avgpool_body,
        out_shape=jax.ShapeDtypeStruct((n, c), jnp.float32),
        grid=(c // tc,),
        in_specs=[pl.BlockSpec((n, hw, tc), lambda j: (0, 0, j))],
        out_specs=pl.BlockSpec((n, tc), lambda j: (0, j)),
        compiler_params=pltpu.CompilerParams(
            dimension_semantics=("parallel",), vmem_limit_bytes=_VMEM),
    )(xr)


_CFG = ((3, 1), (4, 2), (6, 2), (3, 2))


def _bottleneck(x, c1w, c1b, c2w, c2b, c3w, c3b, downw, downb, stride):
    n, h, w, ci = x.shape
    if downw is not None:
        xs = x[:, ::stride, ::stride, :] if stride > 1 else x
        hs, ws = xs.shape[1], xs.shape[2]
        idn = _matmul(xs.reshape(n * hs * ws, ci), downw, downb, relu=False)
    else:
        idn = None
    y = _matmul(x.reshape(n * h * w, ci), c1w, c1b).reshape(n, h, w, -1)
    if stride == 1:
        y = _conv3x3_s1(y, c2w, c2b)
    else:
        y = _conv3x3_s2(y, c2w, c2b)
    n2, h2, w2, cm = y.shape
    co = c3w.shape[1]
    if idn is None:
        idn = x.reshape(n2 * h2 * w2, co)
    y = _matmul(y.reshape(n2 * h2 * w2, cm), c3w, c3b, residual=idn)
    return y.reshape(n2, h2, w2, co)


def kernel(x, stem_w, stem_b, l0b0_c1_w, l0b0_c1_b, l0b0_c2_w, l0b0_c2_b, l0b0_c3_w, l0b0_c3_b, l0b0_down_w, l0b0_down_b, l0b1_c1_w, l0b1_c1_b, l0b1_c2_w, l0b1_c2_b, l0b1_c3_w, l0b1_c3_b, l0b2_c1_w, l0b2_c1_b, l0b2_c2_w, l0b2_c2_b, l0b2_c3_w, l0b2_c3_b, l1b0_c1_w, l1b0_c1_b, l1b0_c2_w, l1b0_c2_b, l1b0_c3_w, l1b0_c3_b, l1b0_down_w, l1b0_down_b, l1b1_c1_w, l1b1_c1_b, l1b1_c2_w, l1b1_c2_b, l1b1_c3_w, l1b1_c3_b, l1b2_c1_w, l1b2_c1_b, l1b2_c2_w, l1b2_c2_b, l1b2_c3_w, l1b2_c3_b, l1b3_c1_w, l1b3_c1_b, l1b3_c2_w, l1b3_c2_b, l1b3_c3_w, l1b3_c3_b, l2b0_c1_w, l2b0_c1_b, l2b0_c2_w, l2b0_c2_b, l2b0_c3_w, l2b0_c3_b, l2b0_down_w, l2b0_down_b, l2b1_c1_w, l2b1_c1_b, l2b1_c2_w, l2b1_c2_b, l2b1_c3_w, l2b1_c3_b, l2b2_c1_w, l2b2_c1_b, l2b2_c2_w, l2b2_c2_b, l2b2_c3_w, l2b2_c3_b, l2b3_c1_w, l2b3_c1_b, l2b3_c2_w, l2b3_c2_b, l2b3_c3_w, l2b3_c3_b, l2b4_c1_w, l2b4_c1_b, l2b4_c2_w, l2b4_c2_b, l2b4_c3_w, l2b4_c3_b, l2b5_c1_w, l2b5_c1_b, l2b5_c2_w, l2b5_c2_b, l2b5_c3_w, l2b5_c3_b, l3b0_c1_w, l3b0_c1_b, l3b0_c2_w, l3b0_c2_b, l3b0_c3_w, l3b0_c3_b, l3b0_down_w, l3b0_down_b, l3b1_c1_w, l3b1_c1_b, l3b1_c2_w, l3b1_c2_b, l3b1_c3_w, l3b1_c3_b, l3b2_c1_w, l3b2_c1_b, l3b2_c2_w, l3b2_c2_b, l3b2_c3_w, l3b2_c3_b):
    prm = dict(locals())
    xh = jnp.transpose(x, (0, 2, 3, 1)).astype(jnp.bfloat16)
    xh = _stem_conv(xh, stem_w, stem_b)
    xh = _maxpool_3x3_s2(xh)
    for li, (blocks, stride) in enumerate(_CFG):
        for bi in range(blocks):
            s = stride if bi == 0 else 1
            pfx = f"l{li}b{bi}_"
            downw = prm.get(pfx + "down_w")
            downb = prm.get(pfx + "down_b")
            xh = _bottleneck(xh,
                             prm[pfx + "c1_w"], prm[pfx + "c1_b"],
                             prm[pfx + "c2_w"], prm[pfx + "c2_b"],
                             prm[pfx + "c3_w"], prm[pfx + "c3_b"],
                             downw, downb, s)
    return _global_avgpool(xh)


# speedup vs baseline: 5.0315x; 1.1422x over previous
import jax
import jax.numpy as jnp
from jax.experimental import pallas as pl
from jax.experimental.pallas import tpu as pltpu

_VMEM = 64 * 1024 * 1024


def _mm_body(a_ref, w_ref, b_ref, o_ref):
    y = jnp.dot(a_ref[...], w_ref[...], preferred_element_type=jnp.float32)
    o_ref[...] = jnp.maximum(y + b_ref[...], 0.0).astype(o_ref.dtype)


def _mm_body_norelu(a_ref, w_ref, b_ref, o_ref):
    y = jnp.dot(a_ref[...], w_ref[...], preferred_element_type=jnp.float32)
    o_ref[...] = (y + b_ref[...]).astype(o_ref.dtype)


def _mm_body_res(a_ref, w_ref, b_ref, r_ref, o_ref):
    y = jnp.dot(a_ref[...], w_ref[...], preferred_element_type=jnp.float32)
    y = y + b_ref[...] + r_ref[...].astype(jnp.float32)
    o_ref[...] = jnp.maximum(y, 0.0).astype(o_ref.dtype)


def _pick_tm(m):
    for c in (512, 448, 256, 224, 128, 112, 64, 56, 32, 16, 8):
        if m % c == 0 and m // c >= 8:
            return c
    for c in (512, 448, 256, 224, 128, 112, 64, 56, 32, 16, 8):
        if m % c == 0:
            return c
    return m


def _matmul(a, w, b, residual=None, relu=True):
    m, k = a.shape
    kp, n = w.shape
    if kp != k:
        a = jnp.pad(a, ((0, 0), (0, kp - k)))
    tm = _pick_tm(m)
    grid = (m // tm,)
    in_specs = [
        pl.BlockSpec((tm, kp), lambda i: (i, 0)),
        pl.BlockSpec((kp, n), lambda i: (0, 0)),
        pl.BlockSpec((1, n), lambda i: (0, 0)),
    ]
    inputs = [a, w, b]
    if residual is not None:
        body = _mm_body_res
        in_specs.append(pl.BlockSpec((tm, n), lambda i: (i, 0)))
        inputs.append(residual)
    else:
        body = _mm_body if relu else _mm_body_norelu
    return pl.pallas_call(
        body,
        out_shape=jax.ShapeDtypeStruct((m, n), jnp.bfloat16),
        grid=grid,
        in_specs=in_specs,
        out_specs=pl.BlockSpec((tm, n), lambda i: (i, 0)),
        compiler_params=pltpu.CompilerParams(
            dimension_semantics=("parallel",), vmem_limit_bytes=_VMEM),
    )(*inputs)


def _c1s2_body(x_ref, w_ref, b_ref, o_ref):
    n, wo, co = o_ref.shape
    c = x_ref.shape[-1] // 2
    a = x_ref[...][:, :, 0:c].reshape(n * wo, c)
    y = jnp.dot(a, w_ref[...], preferred_element_type=jnp.float32)
    o_ref[...] = (y + b_ref[...]).astype(o_ref.dtype).reshape(n, wo, co)


def _conv1x1_s2(x, w, b):
    n, h, wd, c = x.shape
    co = w.shape[1]
    ho, wo = h // 2, wd // 2
    xr = x.reshape(n, h, wo, 2 * c)
    return pl.pallas_call(
        _c1s2_body,
        out_shape=jax.ShapeDtypeStruct((n, ho, wo, co), x.dtype),
        grid=(ho,),
        in_specs=[pl.BlockSpec((n, None, wo, 2 * c), lambda r: (0, 2 * r, 0, 0)),
                  pl.BlockSpec((c, co), lambda r: (0, 0)),
                  pl.BlockSpec((1, co), lambda r: (0, 0))],
        out_specs=pl.BlockSpec((n, None, wo, co), lambda r: (0, r, 0, 0)),
        compiler_params=pltpu.CompilerParams(
            dimension_semantics=("parallel",), vmem_limit_bytes=_VMEM),
    )(xr, w, b)


def _c3s1_body(x0_ref, x1_ref, x2_ref, w_ref, b_ref, o_ref):
    n, wo, co = o_ref.shape
    c = x0_ref.shape[-1]
    acc = jnp.zeros((n * wo, co), jnp.float32)
    for di, xr in enumerate((x0_ref, x1_ref, x2_ref)):
        xv = xr[...]
        for dj in range(3):
            a = xv[:, dj:dj + wo, :].reshape(n * wo, c)
            acc = acc + jnp.dot(a, w_ref[di * 3 + dj],
                                preferred_element_type=jnp.float32)
    y = jnp.maximum(acc + b_ref[...], 0.0)
    o_ref[...] = y.reshape(n, wo, co).astype(o_ref.dtype)


def _conv3x3_s1(x, w9, b):
    n, h, w, c = x.shape
    co = w9.shape[2]
    xp = jnp.pad(x, ((0, 0), (1, 1), (1, 1), (0, 0)))
    wp = w + 2

    def xs(di):
        return pl.BlockSpec((n, None, wp, c), lambda r, di=di: (0, r + di, 0, 0))

    return pl.pallas_call(
        _c3s1_body,
        out_shape=jax.ShapeDtypeStruct((n, h, w, co), x.dtype),
        grid=(h,),
        in_specs=[xs(0), xs(1), xs(2),
                  pl.BlockSpec((9, c, co), lambda r: (0, 0, 0)),
                  pl.BlockSpec((1, co), lambda r: (0, 0))],
        out_specs=pl.BlockSpec((n, None, w, co), lambda r: (0, r, 0, 0)),
        compiler_params=pltpu.CompilerParams(
            dimension_semantics=("parallel",), vmem_limit_bytes=_VMEM),
    )(xp, xp, xp, w9, b)


def _c3s2_body(x0_ref, x1_ref, x2_ref, wa_ref, wb_ref, b_ref, o_ref):
    n, wo, co = o_ref.shape
    c2 = x0_ref.shape[-1]
    c = c2 // 2
    acc = jnp.zeros((n * wo, co), jnp.float32)
    for di, xr in enumerate((x0_ref, x1_ref, x2_ref)):
        xv = xr[...]
        acc = acc + jnp.dot(xv[:, 0:wo, :].reshape(n * wo, c2), wa_ref[di],
                            preferred_element_type=jnp.float32)
        acc = acc + jnp.dot(xv[:, 1:1 + wo, 0:c].reshape(n * wo, c),
                            wb_ref[di], preferred_element_type=jnp.float32)
    y = jnp.maximum(acc + b_ref[...], 0.0)
    o_ref[...] = y.reshape(n, wo, co).astype(o_ref.dtype)


def _conv3x3_s2(x, wflat, b):
    n, h, w, c = x.shape
    co = wflat.shape[1]
    w9 = wflat[:9 * c].reshape(3, 3, c, co)
    wa = w9[:, 0:2].reshape(3, 2 * c, co)
    wb = w9[:, 2]
    ho, wo = h // 2, w // 2
    xp = jnp.pad(x, ((0, 0), (1, 1), (1, 1), (0, 0)))
    wpair = (w + 2) // 2
    xr = xp.reshape(n, h + 2, wpair, 2 * c)

    def xs(di):
        return pl.BlockSpec((n, None, wpair, 2 * c),
                            lambda r, di=di: (0, 2 * r + di, 0, 0))

    return pl.pallas_call(
        _c3s2_body,
        out_shape=jax.ShapeDtypeStruct((n, ho, wo, co), x.dtype),
        grid=(ho,),
        in_specs=[xs(0), xs(1), xs(2),
                  pl.BlockSpec((3, 2 * c, co), lambda r: (0, 0, 0)),
                  pl.BlockSpec((3, c, co), lambda r: (0, 0, 0)),
                  pl.BlockSpec((1, co), lambda r: (0, 0))],
        out_specs=pl.BlockSpec((n, None, wo, co), lambda r: (0, r, 0, 0)),
        compiler_params=pltpu.CompilerParams(
            dimension_semantics=("parallel",), vmem_limit_bytes=_VMEM),
    )(xr, xr, xr, wa, wb, b)


def _stem_body(x0_ref, x1_ref, x2_ref, x3_ref, x4_ref, x5_ref, x6_ref,
               w_ref, b_ref, o_ref):
    n, wo, co = o_ref.shape
    kc = x0_ref.shape[-1]
    acc = jnp.zeros((n * wo, co), jnp.float32)
    for di, xr in enumerate((x0_ref, x1_ref, x2_ref, x3_ref, x4_ref,
                             x5_ref, x6_ref)):
        acc = acc + jnp.dot(xr[...].reshape(n * wo, kc), w_ref[di],
                            preferred_element_type=jnp.float32)
    y = jnp.maximum(acc + b_ref[...], 0.0)
    o_ref[...] = y.reshape(n, wo, co).astype(o_ref.dtype)


def _stem_conv(x_nhwc, stem_w, stem_b):
    n, h, w, _ = x_nhwc.shape
    ho = wo = h // 2
    co = stem_w.shape[1]
    xp = jnp.pad(x_nhwc, ((0, 0), (3, 3), (3, 3), (0, 0)))
    hp = h + 6
    xr = xp.reshape(n, hp, (h + 6) // 2, 6)
    cp = jnp.concatenate([xr[:, :, a:a + wo, :] for a in range(4)], axis=-1)
    wt = jnp.pad(stem_w[:147].reshape(7, 21, co), ((0, 0), (0, 3), (0, 0)))

    def xs(di):
        return pl.BlockSpec((n, None, wo, 24),
                            lambda r, di=di: (0, 2 * r + di, 0, 0))

    return pl.pallas_call(
        _stem_body,
        out_shape=jax.ShapeDtypeStruct((n, ho, wo, co), x_nhwc.dtype),
        grid=(ho,),
        in_specs=[xs(0), xs(1), xs(2), xs(3), xs(4), xs(5), xs(6),
                  pl.BlockSpec((7, 24, co), lambda r: (0, 0, 0)),
                  pl.BlockSpec((1, co), lambda r: (0, 0))],
        out_specs=pl.BlockSpec((n, None, wo, co), lambda r: (0, r, 0, 0)),
        compiler_params=pltpu.CompilerParams(
            dimension_semantics=("parallel",), vmem_limit_bytes=_VMEM),
    )(cp, cp, cp, cp, cp, cp, cp, wt, stem_b)


def _maxpool_body(x0_ref, x1_ref, x2_ref, o_ref):
    n, wo, c = o_ref.shape
    res = None
    for xr in (x0_ref, x1_ref, x2_ref):
        xv = xr[...]
        rm = jnp.maximum(jnp.maximum(xv[:, 0:wo, 0:c], xv[:, 0:wo, c:2 * c]),
                         xv[:, 1:1 + wo, 0:c])
        res = rm if res is None else jnp.maximum(res, rm)
    o_ref[...] = res


def _maxpool_3x3_s2(x):
    n, h, w, c = x.shape
    ho = (h - 1) // 2 + 1
    wo = (w - 1) // 2 + 1
    xp = jnp.pad(x, ((0, 0), (1, 1), (1, 1), (0, 0)),
                 constant_values=-jnp.inf)
    wpair = (w + 2) // 2
    xr = xp.reshape(n, h + 2, wpair, 2 * c)

    def xs(di):
        return pl.BlockSpec((n, None, wpair, 2 * c),
                            lambda r, di=di: (0, 2 * r + di, 0, 0))

    return pl.pallas_call(
        _maxpool_body,
        out_shape=jax.ShapeDtypeStruct((n, ho, wo, c), x.dtype),
        grid=(ho,),
        in_specs=[xs(0), xs(1), xs(2)],
        out_specs=pl.BlockSpec((n, None, wo, c), lambda r: (0, r, 0, 0)),
        compiler_params=pltpu.CompilerParams(
            dimension_semantics=("parallel",), vmem_limit_bytes=_VMEM),
    )(xr, xr, xr)


def _avgpool_body(x_ref, o_ref):
    x = x_ref[...].astype(jnp.float32)
    o_ref[...] = jnp.sum(x, axis=1) * (1.0 / x_ref.shape[1])


def _global_avgpool(x):
    n, h, w, c = x.shape
    hw = h * w
    xr = x.reshape(n, hw, c)
    tc = 512 if c % 512 == 0 else c
    return pl.pallas_call(
        _avgpool_body,
        out_shape=jax.ShapeDtypeStruct((n, c), jnp.float32),
        grid=(c // tc,),
        in_specs=[pl.BlockSpec((n, hw, tc), lambda j: (0, 0, j))],
        out_specs=pl.BlockSpec((n, tc), lambda j: (0, j)),
        compiler_params=pltpu.CompilerParams(
            dimension_semantics=("parallel",), vmem_limit_bytes=_VMEM),
    )(xr)


_CFG = ((3, 1), (4, 2), (6, 2), (3, 2))


def _bottleneck(x, c1w, c1b, c2w, c2b, c3w, c3b, downw, downb, stride):
    n, h, w, ci = x.shape
    if downw is not None:
        if stride > 1:
            co_d = downw.shape[1]
            idn = _conv1x1_s2(x, downw, downb).reshape(-1, co_d)
        else:
            idn = _matmul(x.reshape(n * h * w, ci), downw, downb, relu=False)
    else:
        idn = None
    y = _matmul(x.reshape(n * h * w, ci), c1w, c1b).reshape(n, h, w, -1)
    if stride == 1:
        y = _conv3x3_s1(y, c2w, c2b)
    else:
        y = _conv3x3_s2(y, c2w, c2b)
    n2, h2, w2, cm = y.shape
    co = c3w.shape[1]
    if idn is None:
        idn = x.reshape(n2 * h2 * w2, co)
    y = _matmul(y.reshape(n2 * h2 * w2, cm), c3w, c3b, residual=idn)
    return y.reshape(n2, h2, w2, co)


def kernel(x, stem_w, stem_b, l0b0_c1_w, l0b0_c1_b, l0b0_c2_w, l0b0_c2_b, l0b0_c3_w, l0b0_c3_b, l0b0_down_w, l0b0_down_b, l0b1_c1_w, l0b1_c1_b, l0b1_c2_w, l0b1_c2_b, l0b1_c3_w, l0b1_c3_b, l0b2_c1_w, l0b2_c1_b, l0b2_c2_w, l0b2_c2_b, l0b2_c3_w, l0b2_c3_b, l1b0_c1_w, l1b0_c1_b, l1b0_c2_w, l1b0_c2_b, l1b0_c3_w, l1b0_c3_b, l1b0_down_w, l1b0_down_b, l1b1_c1_w, l1b1_c1_b, l1b1_c2_w, l1b1_c2_b, l1b1_c3_w, l1b1_c3_b, l1b2_c1_w, l1b2_c1_b, l1b2_c2_w, l1b2_c2_b, l1b2_c3_w, l1b2_c3_b, l1b3_c1_w, l1b3_c1_b, l1b3_c2_w, l1b3_c2_b, l1b3_c3_w, l1b3_c3_b, l2b0_c1_w, l2b0_c1_b, l2b0_c2_w, l2b0_c2_b, l2b0_c3_w, l2b0_c3_b, l2b0_down_w, l2b0_down_b, l2b1_c1_w, l2b1_c1_b, l2b1_c2_w, l2b1_c2_b, l2b1_c3_w, l2b1_c3_b, l2b2_c1_w, l2b2_c1_b, l2b2_c2_w, l2b2_c2_b, l2b2_c3_w, l2b2_c3_b, l2b3_c1_w, l2b3_c1_b, l2b3_c2_w, l2b3_c2_b, l2b3_c3_w, l2b3_c3_b, l2b4_c1_w, l2b4_c1_b, l2b4_c2_w, l2b4_c2_b, l2b4_c3_w, l2b4_c3_b, l2b5_c1_w, l2b5_c1_b, l2b5_c2_w, l2b5_c2_b, l2b5_c3_w, l2b5_c3_b, l3b0_c1_w, l3b0_c1_b, l3b0_c2_w, l3b0_c2_b, l3b0_c3_w, l3b0_c3_b, l3b0_down_w, l3b0_down_b, l3b1_c1_w, l3b1_c1_b, l3b1_c2_w, l3b1_c2_b, l3b1_c3_w, l3b1_c3_b, l3b2_c1_w, l3b2_c1_b, l3b2_c2_w, l3b2_c2_b, l3b2_c3_w, l3b2_c3_b):
    prm = dict(locals())
    xh = jnp.transpose(x, (0, 2, 3, 1)).astype(jnp.bfloat16)
    xh = _stem_conv(xh, stem_w, stem_b)
    xh = _maxpool_3x3_s2(xh)
    for li, (blocks, stride) in enumerate(_CFG):
        for bi in range(blocks):
            s = stride if bi == 0 else 1
            pfx = f"l{li}b{bi}_"
            downw = prm.get(pfx + "down_w")
            downb = prm.get(pfx + "down_b")
            xh = _bottleneck(xh,
                             prm[pfx + "c1_w"], prm[pfx + "c1_b"],
                             prm[pfx + "c2_w"], prm[pfx + "c2_b"],
                             prm[pfx + "c3_w"], prm[pfx + "c3_b"],
                             downw, downb, s)
    return _global_avgpool(xh)
